# baseline (speedup 1.0000x reference)
import functools

import jax
import jax.numpy as jnp
from jax.experimental import pallas as pl
from jax.experimental.pallas import tpu as pltpu

BEV_H, BEV_W = 200, 200
RES = 0.5
X0, Y0 = -50.0, -50.0
FEAT = 256
EPS = 1e-5

N_PTS = 20000
CHUNK = 2000
S = 8
N_CHUNKS = N_PTS // CHUNK
ROWS = 1000
LANES = 256


def _scatter_kernel(rh_ref, ri_ref, xi_ref, vh_ref, vi_ref, out_ref, acc):
    step = pl.program_id(0)

    @pl.when(step == 0)
    def _init():
        acc[...] = jnp.zeros_like(acc)

    iota = jax.lax.broadcasted_iota(jnp.int32, (1, LANES), 1)

    def body(i, _):
        for k in range(S):
            idx = i * S + k
            r = rh_ref[0, 0, idx]
            ri = ri_ref[0, 0, idx]
            c = xi_ref[0, 0, idx]
            vh = vh_ref[0, 0, idx]
            vi = vi_ref[0, 0, idx]
            onehot = iota == c
            row = acc[k, pl.ds(r, 1), :]
            acc[k, pl.ds(r, 1), :] = jnp.where(onehot, jnp.maximum(row, vh), row)
            row2 = acc[k, pl.ds(ri, 1), :]
            acc[k, pl.ds(ri, 1), :] = jnp.where(onehot, jnp.maximum(row2, vi), row2)
        return ()

    jax.lax.fori_loop(0, CHUNK // S, body, ())

    @pl.when(step == N_CHUNKS - 1)
    def _reduce():
        m01 = jnp.maximum(acc[0], acc[1])
        m23 = jnp.maximum(acc[2], acc[3])
        m45 = jnp.maximum(acc[4], acc[5])
        m67 = jnp.maximum(acc[6], acc[7])
        out_ref[...] = jnp.maximum(jnp.maximum(m01, m23), jnp.maximum(m45, m67))


def _points_to_bev_pallas(points):
    x, y, z, inten = points[:, 0], points[:, 1], points[:, 2], points[:, 3]
    xi = jnp.clip(jnp.floor((x - X0) / RES).astype(jnp.int32), 0, BEV_W - 1)
    yi = jnp.clip(jnp.floor((y - Y0) / RES).astype(jnp.int32), 0, BEV_H - 1)
    hi = ((z > -2.0).astype(jnp.int32) + (z > 0.0).astype(jnp.int32)
          + (z > 2.0).astype(jnp.int32) + (z > 4.0).astype(jnp.int32))
    hi = jnp.clip(hi, 0, 3)
    rh = (hi * BEV_H + yi).reshape(N_CHUNKS, 1, CHUNK)
    ri = (800 + yi).reshape(N_CHUNKS, 1, CHUNK)
    xi3 = xi.reshape(N_CHUNKS, 1, CHUNK)
    vh = (z + 2.0).reshape(N_CHUNKS, 1, CHUNK)
    vi = inten.reshape(N_CHUNKS, 1, CHUNK)

    smem = pl.BlockSpec((1, 1, CHUNK), lambda i: (i, 0, 0),
                        memory_space=pltpu.SMEM)
    grid_out = pl.pallas_call(
        _scatter_kernel,
        grid=(N_CHUNKS,),
        in_specs=[smem, smem, smem, smem, smem],
        out_specs=pl.BlockSpec((ROWS, LANES), lambda i: (0, 0)),
        out_shape=jax.ShapeDtypeStruct((ROWS, LANES), jnp.float32),
        scratch_shapes=[pltpu.VMEM((S, ROWS, LANES), jnp.float32)],
    )(rh, ri, xi3, vh, vi)
    return grid_out[:, :BEV_W].reshape(1, 5, BEV_H, BEV_W)


def _c2d(x, p, stride=1, pad=1):
    y = jax.lax.conv_general_dilated(
        x, p["w"], (stride, stride), [(pad, pad), (pad, pad)],
        dimension_numbers=("NCHW", "OIHW", "NCHW"))
    return y + p["b"][None, :, None, None]


def _bnorm(x, p):
    s = p["g"] * jax.lax.rsqrt(p["v"] + EPS)
    return x * s[None, :, None, None] + (p["beta"] - p["m"] * s)[None, :, None, None]


def _cam_branch(img, p):
    f = jax.nn.relu(_bnorm(_c2d(img, p["c1"], stride=2, pad=1), p["bn1"]))
    f = jax.nn.relu(_bnorm(_c2d(f, p["c2"], stride=2, pad=1), p["bn2"]))
    f = jax.nn.relu(_bnorm(_c2d(f, p["c3"], stride=2, pad=1), p["bn3"]))
    f = jax.nn.relu(_bnorm(_c2d(f, p["p1"], stride=1, pad=1), p["pbn"]))
    f = _c2d(f, p["p2"], stride=1, pad=0)
    return jax.image.resize(f, (f.shape[0], f.shape[1], BEV_H, BEV_W),
                            method="bilinear", antialias=False)


def _lid_branch(points, p):
    bev = _points_to_bev_pallas(points)
    f = jax.nn.relu(_bnorm(_c2d(bev, p["c1"], stride=1, pad=1), p["bn1"]))
    f = jax.nn.relu(_bnorm(_c2d(f, p["c2"], stride=1, pad=1), p["bn2"]))
    return _c2d(f, p["c3"], stride=1, pad=0)


def kernel(images, points, cam_params, lidar_params):
    cam = _cam_branch(images, cam_params)
    lid = _lid_branch(points, lidar_params)
    return jnp.concatenate([cam, lid], axis=1)
